# baseline (speedup 1.0000x reference)
import functools

import jax
import jax.numpy as jnp
from jax import lax
from jax.experimental import pallas as pl
from jax.experimental.pallas import tpu as pltpu
from jax.experimental.pallas import tpu_sc as plsc


def _table_matmul(emb_table, W, b2d):
    V, D = emb_table.shape
    Vout = W.shape[1]

    def body(emb_ref, w_ref, b_ref, out_ref):
        out_ref[...] = (
            jnp.dot(emb_ref[...], w_ref[...], preferred_element_type=jnp.float32)
            + b_ref[...]
        )

    return pl.pallas_call(
        body,
        out_shape=jax.ShapeDtypeStruct((V, Vout), jnp.float32),
    )(emb_table, W, b2d)


@functools.lru_cache(maxsize=None)
def _make_gather(N, V, CB):
    info = plsc.get_sparse_core_info()
    NC, NS = info.num_cores, info.num_subcores
    NW = NC * NS
    assert N % (NW * CB) == 0
    per_w = N // NW
    n_chunks = per_w // CB

    mesh = plsc.VectorSubcoreMesh(core_axis_name="c", subcore_axis_name="s")

    @functools.partial(
        pl.kernel,
        out_type=jax.ShapeDtypeStruct((N, V), jnp.float32),
        mesh=mesh,
        compiler_params=pltpu.CompilerParams(use_tc_tiling_on_sc=False),
        scratch_types=[
            pltpu.VMEM((per_w,), jnp.int32),
            pltpu.VMEM((CB, V), jnp.float32),
            pltpu.SemaphoreType.DMA,
        ],
    )
    def gather(table_hbm, ids_hbm, out_hbm, idx_v, rows_v, sem):
        wid = lax.axis_index("s") * NC + lax.axis_index("c")
        base = wid * per_w
        pltpu.sync_copy(ids_hbm.at[pl.ds(base, per_w)], idx_v)

        def step(i, carry):
            pltpu.async_copy(
                table_hbm.at[idx_v.at[pl.ds(i * CB, CB)]], rows_v, sem
            ).wait()
            pltpu.sync_copy(rows_v, out_hbm.at[pl.ds(base + i * CB, CB)])
            return carry

        lax.fori_loop(0, n_chunks, step, 0)

    return gather


def kernel(input_ids, emb_table, W, b):
    Bsz, T = input_ids.shape
    V = W.shape[1]
    N = Bsz * T

    table = _table_matmul(emb_table, W, b.reshape(1, V))
    ids = input_ids.reshape(N).astype(jnp.int32)
    out = _make_gather(N, V, 64)(table, ids)
    return out.reshape(Bsz, T, V)


# speedup vs baseline: 1.0127x; 1.0127x over previous
import functools

import jax
import jax.numpy as jnp
from jax import lax
from jax.experimental import pallas as pl
from jax.experimental.pallas import tpu as pltpu
from jax.experimental.pallas import tpu_sc as plsc


def _table_matmul(emb_table, W, b2d):
    V, D = emb_table.shape
    Vout = W.shape[1]

    def body(emb_ref, w_ref, b_ref, out_ref):
        out_ref[...] = (
            jnp.dot(emb_ref[...], w_ref[...], preferred_element_type=jnp.float32)
            + b_ref[...]
        )

    return pl.pallas_call(
        body,
        out_shape=jax.ShapeDtypeStruct((V, Vout), jnp.float32),
    )(emb_table, W, b2d)


@functools.lru_cache(maxsize=None)
def _make_gather(N, V, CB):
    info = plsc.get_sparse_core_info()
    NC, NS = info.num_cores, info.num_subcores
    NW = NC * NS
    assert N % (NW * CB) == 0
    per_w = N // NW
    n_chunks = per_w // CB
    assert n_chunks % 2 == 0 and n_chunks >= 4
    P = n_chunks // 2

    mesh = plsc.VectorSubcoreMesh(core_axis_name="c", subcore_axis_name="s")

    @functools.partial(
        pl.kernel,
        out_type=jax.ShapeDtypeStruct((N, V), jnp.float32),
        mesh=mesh,
        compiler_params=pltpu.CompilerParams(use_tc_tiling_on_sc=False),
        scratch_types=[
            pltpu.VMEM((per_w,), jnp.int32),
            pltpu.VMEM((CB, V), jnp.float32),
            pltpu.VMEM((CB, V), jnp.float32),
            pltpu.SemaphoreType.DMA,
            pltpu.SemaphoreType.DMA,
            pltpu.SemaphoreType.DMA,
            pltpu.SemaphoreType.DMA,
        ],
    )
    def gather(table_hbm, ids_hbm, out_hbm, idx_v, r0, r1, g0, g1, s0, s1):
        wid = lax.axis_index("s") * NC + lax.axis_index("c")
        base = wid * per_w
        pltpu.sync_copy(ids_hbm.at[pl.ds(base, per_w)], idx_v)

        def g_start(chunk, buf, sem):
            pltpu.async_copy(table_hbm.at[idx_v.at[pl.ds(chunk * CB, CB)]], buf, sem)

        def g_wait(buf, sem):
            pltpu.make_async_copy(
                table_hbm.at[idx_v.at[pl.ds(0, CB)]], buf, sem
            ).wait()

        def s_start(chunk, buf, sem):
            pltpu.async_copy(buf, out_hbm.at[pl.ds(base + chunk * CB, CB)], sem)

        def s_wait(buf, sem):
            pltpu.make_async_copy(buf, out_hbm.at[pl.ds(base, CB)], sem).wait()

        g_start(0, r0, g0)
        g_start(1, r1, g1)
        g_wait(r0, g0)
        s_start(0, r0, s0)
        s_wait(r0, s0)
        g_start(2, r0, g0)
        g_wait(r1, g1)
        s_start(1, r1, s1)

        last = n_chunks - 1

        def body(i, carry):
            a = 2 * i
            s_wait(r1, s1)
            g_start(a + 1, r1, g1)
            g_wait(r0, g0)
            s_start(a, r0, s0)
            s_wait(r0, s0)
            g_start(jnp.minimum(a + 2, last), r0, g0)
            g_wait(r1, g1)
            s_start(a + 1, r1, s1)
            return carry

        lax.fori_loop(1, P, body, 0)

        s_wait(r1, s1)
        g_wait(r0, g0)

    return gather


def kernel(input_ids, emb_table, W, b):
    Bsz, T = input_ids.shape
    V = W.shape[1]
    N = Bsz * T

    table = _table_matmul(emb_table, W, b.reshape(1, V))
    ids = input_ids.reshape(N).astype(jnp.int32)
    out = _make_gather(N, V, 32)(table, ids)
    return out.reshape(Bsz, T, V)


# speedup vs baseline: 2.0094x; 1.9842x over previous
import functools

import jax
import jax.numpy as jnp
from jax import lax
from jax.experimental import pallas as pl
from jax.experimental.pallas import tpu as pltpu
from jax.experimental.pallas import tpu_sc as plsc

_TPAD = 56


@functools.lru_cache(maxsize=None)
def _make_sc_lookup(V, D, NPAD):
    info = plsc.get_sparse_core_info()
    NC, NS, L = info.num_cores, info.num_subcores, info.num_lanes
    NW = NC * NS
    assert D == L and NPAD % (NW * L) == 0
    per_w = NPAD // NW
    n_grp = per_w // L

    mesh = plsc.VectorSubcoreMesh(core_axis_name="c", subcore_axis_name="s")

    @functools.partial(
        pl.kernel,
        out_type=jax.ShapeDtypeStruct((D, NPAD), jnp.float32),
        mesh=mesh,
        compiler_params=pltpu.CompilerParams(use_tc_tiling_on_sc=False, needs_layout_passes=False),
        scratch_types=[
            pltpu.VMEM((V * D,), jnp.float32),
            pltpu.VMEM((per_w,), jnp.int32),
            pltpu.VMEM((D, per_w), jnp.float32),
        ],
    )
    def lookup(emb_hbm, ids_hbm, xt_hbm, emb_v, idx_v, tbuf):
        wid = lax.axis_index("s") * NC + lax.axis_index("c")
        base = wid * per_w
        pltpu.sync_copy(emb_hbm, emb_v)
        pltpu.sync_copy(ids_hbm.at[pl.ds(base, per_w)], idx_v)

        def grp(g, carry):
            col = g * L
            idx_g = idx_v[pl.ds(col, L)]
            for d in range(D):
                tbuf[d, pl.ds(col, L)] = plsc.load_gather(
                    emb_v, [idx_g * D + d])
            return carry

        lax.fori_loop(0, n_grp, grp, 0)
        pltpu.sync_copy(tbuf, xt_hbm.at[:, pl.ds(base, per_w)])

    return lookup


@functools.lru_cache(maxsize=None)
def _make_tc_head(Bsz, T, V, D, BB):
    assert Bsz % BB == 0
    cols = BB * _TPAD

    def body(xt_ref, w_ref, b_ref, out_ref):
        res = lax.dot_general(
            xt_ref[...], w_ref[...],
            dimension_numbers=(((0,), (0,)), ((), ())),
            preferred_element_type=jnp.float32,
        ) + b_ref[...]
        for j in range(BB):
            out_ref[j] = res[j * _TPAD:j * _TPAD + T, :]

    return pl.pallas_call(
        body,
        grid=(Bsz // BB,),
        in_specs=[
            pl.BlockSpec((D, cols), lambda i: (0, i)),
            pl.BlockSpec((D, V), lambda i: (0, 0)),
            pl.BlockSpec((1, V), lambda i: (0, 0)),
        ],
        out_specs=pl.BlockSpec((BB, T, V), lambda i: (i, 0, 0)),
        out_shape=jax.ShapeDtypeStruct((Bsz, T, V), jnp.float32),
    )


def kernel(input_ids, emb_table, W, b):
    Bsz, T = input_ids.shape
    V, D = emb_table.shape
    Vout = W.shape[1]
    NPAD = Bsz * _TPAD

    ids_pad = jnp.pad(input_ids.astype(jnp.int32), ((0, 0), (0, _TPAD - T)))
    xt = _make_sc_lookup(V, D, NPAD)(
        emb_table.reshape(V * D), ids_pad.reshape(NPAD))
    return _make_tc_head(Bsz, T, Vout, D, 16)(xt, W, b.reshape(1, Vout))


# speedup vs baseline: 5.9396x; 2.9559x over previous
import functools

import jax
import jax.numpy as jnp
from jax import lax
from jax.experimental import pallas as pl
from jax.experimental.pallas import tpu as pltpu
from jax.experimental.pallas import tpu_sc as plsc


@functools.lru_cache(maxsize=None)
def _make_sc_lookup(V, D, T, B):
    info = plsc.get_sparse_core_info()
    NC, NS, L = info.num_cores, info.num_subcores, info.num_lanes
    NW = NC * NS
    assert D == L and B % (NW * L) == 0
    bw = B // NW
    n_h = bw // L

    mesh = plsc.VectorSubcoreMesh(core_axis_name="c", subcore_axis_name="s")

    @functools.partial(
        pl.kernel,
        out_type=jax.ShapeDtypeStruct((T, D, B), jnp.float32),
        mesh=mesh,
        compiler_params=pltpu.CompilerParams(
            use_tc_tiling_on_sc=False, needs_layout_passes=False
        ),
        scratch_types=[
            pltpu.VMEM((V * D,), jnp.float32),
            pltpu.VMEM((T, bw), jnp.int32),
            pltpu.VMEM((T, D, bw), jnp.float32),
        ],
    )
    def lookup(emb_hbm, ids_hbm, xt_hbm, emb_v, idx_v, tbuf):
        wid = lax.axis_index("s") * NC + lax.axis_index("c")
        bcol = wid * bw
        pltpu.sync_copy(emb_hbm, emb_v)
        pltpu.sync_copy(ids_hbm.at[:, pl.ds(bcol, bw)], idx_v)

        def tok(t, carry):
            for h in range(n_h):
                idx_g = idx_v[t, pl.ds(h * L, L)]
                for d in range(D):
                    tbuf[t, d, pl.ds(h * L, L)] = plsc.load_gather(
                        emb_v, [idx_g * D + d]
                    )
            return carry

        lax.fori_loop(0, T, tok, 0)
        pltpu.sync_copy(tbuf, xt_hbm.at[:, :, pl.ds(bcol, bw)])

    return lookup


@functools.lru_cache(maxsize=None)
def _make_tc_head(T, V, D, B):
    def body(xt_ref, w_ref, b_ref, out_ref):
        out_ref[0] = (
            lax.dot_general(
                w_ref[...], xt_ref[0],
                dimension_numbers=(((0,), (0,)), ((), ())),
                preferred_element_type=jnp.float32,
            )
            + b_ref[...]
        )

    return pl.pallas_call(
        body,
        grid=(T,),
        in_specs=[
            pl.BlockSpec((1, D, B), lambda i: (i, 0, 0)),
            pl.BlockSpec((D, V), lambda i: (0, 0)),
            pl.BlockSpec((V, 1), lambda i: (0, 0)),
        ],
        out_specs=pl.BlockSpec((1, V, B), lambda i: (i, 0, 0)),
        out_shape=jax.ShapeDtypeStruct((T, V, B), jnp.float32),
    )


def kernel(input_ids, emb_table, W, b):
    Bsz, T = input_ids.shape
    V, D = emb_table.shape
    Vout = W.shape[1]

    ids_t = input_ids.astype(jnp.int32).T
    xt3 = _make_sc_lookup(V, D, T, Bsz)(emb_table.reshape(V * D), ids_t)
    tvb = _make_tc_head(T, Vout, D, Bsz)(xt3, W, b.reshape(Vout, 1))
    return jnp.transpose(tvb, (2, 0, 1))


# speedup vs baseline: 6.4369x; 1.0837x over previous
import functools

import jax
import jax.numpy as jnp
from jax import lax
from jax.experimental import pallas as pl
from jax.experimental.pallas import tpu as pltpu
from jax.experimental.pallas import tpu_sc as plsc


@functools.lru_cache(maxsize=None)
def _make_sc_lookup(V, D, T, B):
    info = plsc.get_sparse_core_info()
    NC, NS, L = info.num_cores, info.num_subcores, info.num_lanes
    NW = NC * NS
    assert D == L and B % (NW * L) == 0
    bw = B // NW
    n_h = bw // L

    mesh = plsc.VectorSubcoreMesh(core_axis_name="c", subcore_axis_name="s")

    @functools.partial(
        pl.kernel,
        out_type=jax.ShapeDtypeStruct((T, D, B), jnp.float32),
        mesh=mesh,
        compiler_params=pltpu.CompilerParams(
            use_tc_tiling_on_sc=False, needs_layout_passes=False
        ),
        scratch_types=[
            pltpu.VMEM((V * D,), jnp.float32),
            pltpu.VMEM((T, bw), jnp.int32),
            pltpu.VMEM((T, D, bw), jnp.float32),
        ],
    )
    def lookup(emb_hbm, ids_hbm, xt_hbm, emb_v, idx_v, tbuf):
        wid = lax.axis_index("s") * NC + lax.axis_index("c")
        bcol = wid * bw
        pltpu.sync_copy(emb_hbm, emb_v)
        pltpu.sync_copy(ids_hbm.at[:, pl.ds(bcol, bw)], idx_v)

        @plsc.parallel_loop(0, T, unroll=2)
        def tok(t):
            for h in range(n_h):
                idx_g = idx_v[t, pl.ds(h * L, L)]
                for d in range(D):
                    tbuf[t, d, pl.ds(h * L, L)] = plsc.load_gather(
                        emb_v, [idx_g * D + d]
                    )
        pltpu.sync_copy(tbuf, xt_hbm.at[:, :, pl.ds(bcol, bw)])

    return lookup


@functools.lru_cache(maxsize=None)
def _make_tc_head(T, V, D, B):
    def body(xt_ref, w_ref, b_ref, out_ref):
        out_ref[0] = (
            lax.dot_general(
                w_ref[...], xt_ref[0],
                dimension_numbers=(((0,), (0,)), ((), ())),
                preferred_element_type=jnp.float32,
            )
            + b_ref[...]
        )

    return pl.pallas_call(
        body,
        grid=(T,),
        in_specs=[
            pl.BlockSpec((1, D, B), lambda i: (i, 0, 0)),
            pl.BlockSpec((D, V), lambda i: (0, 0)),
            pl.BlockSpec((V, 1), lambda i: (0, 0)),
        ],
        out_specs=pl.BlockSpec((1, V, B), lambda i: (i, 0, 0)),
        out_shape=jax.ShapeDtypeStruct((T, V, B), jnp.float32),
    )


def kernel(input_ids, emb_table, W, b):
    Bsz, T = input_ids.shape
    V, D = emb_table.shape
    Vout = W.shape[1]

    ids_t = input_ids.astype(jnp.int32).T
    xt3 = _make_sc_lookup(V, D, T, Bsz)(emb_table.reshape(V * D), ids_t)
    tvb = _make_tc_head(T, Vout, D, Bsz)(xt3, W, b.reshape(Vout, 1))
    return jnp.transpose(tvb, (2, 0, 1))
